# baseline (speedup 1.0000x reference)
import jax
import jax.numpy as jnp
from jax.experimental import pallas as pl
from jax.experimental.pallas import tpu as pltpu

_B, _N, _H, _O, _NBR, _HEADS = 1024, 64, 256, 128, 8, 3
_EPS = 1e-5
_Bb1 = 1024
_Bb2 = 128


def _mmlp_kernel(obs_ref, w1_ref, b1_ref, g1_ref, be1_ref, w2_ref, b2_ref,
                 q_ref):
    x = obs_ref[...].reshape(_Bb1, _H).astype(jnp.bfloat16)
    w1 = w1_ref[...].reshape(_H, _H).astype(jnp.bfloat16)
    h = jnp.dot(x, w1, preferred_element_type=jnp.float32) + b1_ref[...].reshape(1, _H)
    mu = jnp.mean(h, axis=1, keepdims=True)
    hc = h - mu
    var = jnp.mean(hc * hc, axis=1, keepdims=True)
    h = (hc * jax.lax.rsqrt(var + _EPS) * g1_ref[...].reshape(1, _H)
         + be1_ref[...].reshape(1, _H))
    h = jnp.maximum(h, 0.0).astype(jnp.bfloat16)
    w2 = w2_ref[...].reshape(_H, _O).astype(jnp.bfloat16)
    q = jnp.dot(h, w2, preferred_element_type=jnp.float32) + b2_ref[...].reshape(1, _O)
    q_ref[...] = q.astype(jnp.bfloat16).reshape(_Bb1, 1, 1, _O)


def _agg_kernel(q_ref, edges_ref, s24_ref, p192_ref, out_ref, a_s, c_s, d_s):
    nb2 = _NBR * _NBR

    @pl.when(pl.program_id(0) == 0)
    def _build_coeffs():
        s_eff = (s24_ref[:, 0:_NBR] + s24_ref[:, _NBR:2 * _NBR]
                 + s24_ref[:, 2 * _NBR:3 * _NBR]) * (1.0 / _HEADS)
        p_mean = (p192_ref[:, 0:nb2] + p192_ref[:, nb2:2 * nb2]
                  + p192_ref[:, 2 * nb2:3 * nb2]) * (1.0 / _HEADS)
        ij = jax.lax.broadcasted_iota(jnp.int32, (_N, nb2), 1)
        tri = ((ij // _NBR) < (ij % _NBR)).astype(jnp.float32)
        p_eff = p_mean * tri
        l_i = jax.lax.broadcasted_iota(jnp.int32, (nb2, _NBR), 0)
        k_i = jax.lax.broadcasted_iota(jnp.int32, (nb2, _NBR), 1)
        rmask = ((l_i // _NBR) == k_i).astype(jnp.float32)
        cmask = ((l_i % _NBR) == k_i).astype(jnp.float32)
        rowsum = jnp.dot(p_eff, rmask, preferred_element_type=jnp.float32)
        colsum = jnp.dot(p_eff, cmask, preferred_element_type=jnp.float32)
        w = s_eff + 0.5 * (rowsum + colsum)
        m_iota = jax.lax.broadcasted_iota(jnp.int32, (_N, _N), 1)
        acc_a = (edges_ref[:, 0:1] == m_iota).astype(jnp.float32)
        for k in range(_NBR):
            hit = (edges_ref[:, k + 1:k + 2] == m_iota).astype(jnp.float32)
            acc_a = acc_a + w[:, k:k + 1] * hit
        a_s[...] = acc_a
        c_iota = jax.lax.broadcasted_iota(jnp.int32, (_N, _N * _N), 1)
        acc_c = jnp.zeros((_N, _N * _N), jnp.float32)
        for i in range(_NBR):
            for j in range(i + 1, _NBR):
                rowidx = edges_ref[:, i + 1:i + 2] * _N + edges_ref[:, j + 1:j + 2]
                val = p_eff[:, i * _NBR + j:i * _NBR + j + 1] * (-0.5)
                acc_c = acc_c + val * (rowidx == c_iota).astype(jnp.float32)
        c_s[...] = acc_c

    q = q_ref[...]
    for a2 in range(_N // 2):
        qa0 = q[:, 2 * a2:2 * a2 + 1, :]
        qa1 = q[:, 2 * a2 + 1:2 * a2 + 2, :]
        d0 = jnp.sum(jnp.abs(q - qa0).astype(jnp.float32), axis=2)
        d1 = jnp.sum(jnp.abs(q - qa1).astype(jnp.float32), axis=2)
        d_s[:, a2 * 2 * _N:(a2 + 1) * 2 * _N] = jnp.concatenate([d0, d1], axis=1)

    t = jnp.sum(q.astype(jnp.float32), axis=2)
    acc = jax.lax.dot_general(t, a_s[...], (((1,), (1,)), ((), ())),
                              preferred_element_type=jnp.float32)
    acc = acc + jax.lax.dot_general(d_s[...], c_s[...], (((1,), (1,)), ((), ())),
                                    preferred_element_type=jnp.float32)
    out_ref[...] = acc * (1.0 / _O)


def kernel(observation, local_edges, W1, b1, g1, be1, W2, b2, singles, pairs):
    edges = local_edges[:, 0, :].astype(jnp.int32)
    s24 = singles.reshape(_N, _HEADS * _NBR)
    p192 = pairs.reshape(_N, _HEADS * _NBR * _NBR)

    obs4 = observation.reshape(_B, _N, 1, _H)
    q = pl.pallas_call(
        _mmlp_kernel,
        grid=(_N, _B // _Bb1),
        in_specs=[
            pl.BlockSpec((_Bb1, 1, 1, _H), lambda n, bi: (bi, n, 0, 0)),
            pl.BlockSpec((1, _H, _H), lambda n, bi: (n, 0, 0)),
            pl.BlockSpec((1, 1, _H), lambda n, bi: (n, 0, 0)),
            pl.BlockSpec((1, 1, _H), lambda n, bi: (n, 0, 0)),
            pl.BlockSpec((1, 1, _H), lambda n, bi: (n, 0, 0)),
            pl.BlockSpec((1, _H, _O), lambda n, bi: (n, 0, 0)),
            pl.BlockSpec((1, 1, _O), lambda n, bi: (n, 0, 0)),
        ],
        out_specs=pl.BlockSpec((_Bb1, 1, 1, _O), lambda n, bi: (bi, n, 0, 0)),
        out_shape=jax.ShapeDtypeStruct((_B, _N, 1, _O), jnp.bfloat16),
    )(obs4, W1, b1.reshape(_N, 1, _H), g1.reshape(_N, 1, _H),
      be1.reshape(_N, 1, _H), W2, b2.reshape(_N, 1, _O))
    q = q.reshape(_B, _N, _O)

    out = pl.pallas_call(
        _agg_kernel,
        grid=(_B // _Bb2,),
        in_specs=[
            pl.BlockSpec((_Bb2, _N, _O), lambda i: (i, 0, 0)),
            pl.BlockSpec((_N, _NBR + 1), lambda i: (0, 0)),
            pl.BlockSpec((_N, _HEADS * _NBR), lambda i: (0, 0)),
            pl.BlockSpec((_N, _HEADS * _NBR * _NBR), lambda i: (0, 0)),
        ],
        out_specs=pl.BlockSpec((_Bb2, _N), lambda i: (i, 0)),
        out_shape=jax.ShapeDtypeStruct((_B, _N), jnp.float32),
        scratch_shapes=[
            pltpu.VMEM((_N, _N), jnp.float32),
            pltpu.VMEM((_N, _N * _N), jnp.float32),
            pltpu.VMEM((_Bb2, _N * _N), jnp.float32),
        ],
    )(q, edges, s24, p192)
    return out


# speedup vs baseline: 1.3552x; 1.3552x over previous
import jax
import jax.numpy as jnp
from jax.experimental import pallas as pl
from jax.experimental.pallas import tpu as pltpu

_B, _N, _H, _O, _NBR, _HEADS = 1024, 64, 256, 128, 8, 3
_EPS = 1e-5
_Bb1 = 1024
_Bb2 = 128


def _mmlp_kernel(obs_ref, w1_ref, b1_ref, g1_ref, be1_ref, w2_ref, b2_ref,
                 q_ref):
    x = obs_ref[...].reshape(_Bb1, _H).astype(jnp.bfloat16)
    w1 = w1_ref[...].reshape(_H, _H).astype(jnp.bfloat16)
    h = jnp.dot(x, w1, preferred_element_type=jnp.float32) + b1_ref[...].reshape(1, _H)
    mu = jnp.mean(h, axis=1, keepdims=True)
    hc = h - mu
    var = jnp.mean(hc * hc, axis=1, keepdims=True)
    h = (hc * jax.lax.rsqrt(var + _EPS) * g1_ref[...].reshape(1, _H)
         + be1_ref[...].reshape(1, _H))
    h = jnp.maximum(h, 0.0).astype(jnp.bfloat16)
    w2 = w2_ref[...].reshape(_H, _O).astype(jnp.bfloat16)
    q = jnp.dot(h, w2, preferred_element_type=jnp.float32) + b2_ref[...].reshape(1, _O)
    q_ref[...] = q.astype(jnp.bfloat16).reshape(_Bb1, 1, 1, _O)


def _agg_kernel(q_ref, edges_ref, s24_ref, p192_ref, out_ref, a_s, c_s):
    nb2 = _NBR * _NBR

    @pl.when(pl.program_id(0) == 0)
    def _build_coeffs():
        s_eff = (s24_ref[:, 0:_NBR] + s24_ref[:, _NBR:2 * _NBR]
                 + s24_ref[:, 2 * _NBR:3 * _NBR]) * (1.0 / _HEADS)
        p_mean = (p192_ref[:, 0:nb2] + p192_ref[:, nb2:2 * nb2]
                  + p192_ref[:, 2 * nb2:3 * nb2]) * (1.0 / _HEADS)
        ij = jax.lax.broadcasted_iota(jnp.int32, (_N, nb2), 1)
        tri = ((ij // _NBR) < (ij % _NBR)).astype(jnp.float32)
        p_eff = p_mean * tri
        l_i = jax.lax.broadcasted_iota(jnp.int32, (nb2, _NBR), 0)
        k_i = jax.lax.broadcasted_iota(jnp.int32, (nb2, _NBR), 1)
        rmask = ((l_i // _NBR) == k_i).astype(jnp.float32)
        cmask = ((l_i % _NBR) == k_i).astype(jnp.float32)
        rowsum = jnp.dot(p_eff, rmask, preferred_element_type=jnp.float32)
        colsum = jnp.dot(p_eff, cmask, preferred_element_type=jnp.float32)
        w = s_eff + 0.5 * (rowsum + colsum)
        m_iota = jax.lax.broadcasted_iota(jnp.int32, (_N, _N), 1)
        acc_a = (edges_ref[:, 0:1] == m_iota).astype(jnp.float32)
        for k in range(_NBR):
            hit = (edges_ref[:, k + 1:k + 2] == m_iota).astype(jnp.float32)
            acc_a = acc_a + w[:, k:k + 1] * hit
        a_s[...] = acc_a
        c_iota = jax.lax.broadcasted_iota(jnp.int32, (_N, _N * _N), 1)
        acc_c = jnp.zeros((_N, _N * _N), jnp.float32)
        for i in range(_NBR):
            for j in range(i + 1, _NBR):
                ei = edges_ref[:, i + 1:i + 2]
                ej = edges_ref[:, j + 1:j + 2]
                rowidx = jnp.minimum(ei, ej) * _N + jnp.maximum(ei, ej)
                val = p_eff[:, i * _NBR + j:i * _NBR + j + 1] * (-0.5)
                acc_c = acc_c + val * (rowidx == c_iota).astype(jnp.float32)
        c_s[...] = acc_c

    q = q_ref[...]
    t = jnp.sum(q.astype(jnp.float32), axis=2)
    acc = jax.lax.dot_general(t, a_s[...], (((1,), (1,)), ((), ())),
                              preferred_element_type=jnp.float32)
    cmat = c_s[...]
    for a2 in range(_N // 2):
        lo = 2 * a2
        lo8 = (lo // 8) * 8
        qa0 = q[:, lo:lo + 1, :]
        qa1 = q[:, lo + 1:lo + 2, :]
        qc = q[:, lo8:, :]
        d0 = jnp.sum(jnp.abs(qc - qa0).astype(jnp.float32), axis=2)
        d1 = jnp.sum(jnp.abs(qc - qa1).astype(jnp.float32), axis=2)
        dd = jnp.concatenate([d0, d1], axis=1)
        cslice = jnp.concatenate(
            [cmat[:, lo * _N + lo8:(lo + 1) * _N],
             cmat[:, (lo + 1) * _N + lo8:(lo + 2) * _N]], axis=1)
        acc = acc + jax.lax.dot_general(dd, cslice, (((1,), (1,)), ((), ())),
                                        preferred_element_type=jnp.float32)
    out_ref[...] = acc * (1.0 / _O)


def kernel(observation, local_edges, W1, b1, g1, be1, W2, b2, singles, pairs):
    edges = local_edges[:, 0, :].astype(jnp.int32)
    s24 = singles.reshape(_N, _HEADS * _NBR)
    p192 = pairs.reshape(_N, _HEADS * _NBR * _NBR)

    obs4 = observation.reshape(_B, _N, 1, _H)
    q = pl.pallas_call(
        _mmlp_kernel,
        grid=(_N, _B // _Bb1),
        in_specs=[
            pl.BlockSpec((_Bb1, 1, 1, _H), lambda n, bi: (bi, n, 0, 0)),
            pl.BlockSpec((1, _H, _H), lambda n, bi: (n, 0, 0)),
            pl.BlockSpec((1, 1, _H), lambda n, bi: (n, 0, 0)),
            pl.BlockSpec((1, 1, _H), lambda n, bi: (n, 0, 0)),
            pl.BlockSpec((1, 1, _H), lambda n, bi: (n, 0, 0)),
            pl.BlockSpec((1, _H, _O), lambda n, bi: (n, 0, 0)),
            pl.BlockSpec((1, 1, _O), lambda n, bi: (n, 0, 0)),
        ],
        out_specs=pl.BlockSpec((_Bb1, 1, 1, _O), lambda n, bi: (bi, n, 0, 0)),
        out_shape=jax.ShapeDtypeStruct((_B, _N, 1, _O), jnp.bfloat16),
    )(obs4, W1, b1.reshape(_N, 1, _H), g1.reshape(_N, 1, _H),
      be1.reshape(_N, 1, _H), W2, b2.reshape(_N, 1, _O))
    q = q.reshape(_B, _N, _O)

    out = pl.pallas_call(
        _agg_kernel,
        grid=(_B // _Bb2,),
        in_specs=[
            pl.BlockSpec((_Bb2, _N, _O), lambda i: (i, 0, 0)),
            pl.BlockSpec((_N, _NBR + 1), lambda i: (0, 0)),
            pl.BlockSpec((_N, _HEADS * _NBR), lambda i: (0, 0)),
            pl.BlockSpec((_N, _HEADS * _NBR * _NBR), lambda i: (0, 0)),
        ],
        out_specs=pl.BlockSpec((_Bb2, _N), lambda i: (i, 0)),
        out_shape=jax.ShapeDtypeStruct((_B, _N), jnp.float32),
        scratch_shapes=[
            pltpu.VMEM((_N, _N), jnp.float32),
            pltpu.VMEM((_N, _N * _N), jnp.float32),
        ],
    )(q, edges, s24, p192)
    return out


# speedup vs baseline: 1.4261x; 1.0523x over previous
import jax
import jax.numpy as jnp
from jax.experimental import pallas as pl
from jax.experimental.pallas import tpu as pltpu

_B, _N, _H, _O, _NBR, _HEADS = 1024, 64, 256, 128, 8, 3
_EPS = 1e-5
_Bb1 = 1024
_Bb2 = 128


def _mmlp_kernel(obs_ref, w1_ref, b1_ref, g1_ref, be1_ref, w2_ref, b2_ref,
                 q_ref):
    x = obs_ref[...].reshape(_Bb1, _H).astype(jnp.bfloat16)
    w1 = w1_ref[...].reshape(_H, _H).astype(jnp.bfloat16)
    h = jnp.dot(x, w1, preferred_element_type=jnp.float32) + b1_ref[...].reshape(1, _H)
    mu = jnp.mean(h, axis=1, keepdims=True)
    hc = h - mu
    var = jnp.mean(hc * hc, axis=1, keepdims=True)
    h = (hc * jax.lax.rsqrt(var + _EPS) * g1_ref[...].reshape(1, _H)
         + be1_ref[...].reshape(1, _H))
    h = jnp.maximum(h, 0.0).astype(jnp.bfloat16)
    w2 = w2_ref[...].reshape(_H, _O).astype(jnp.bfloat16)
    q = jnp.dot(h, w2, preferred_element_type=jnp.float32) + b2_ref[...].reshape(1, _O)
    q_ref[...] = q.reshape(_Bb1, 1, 1, _O)


def _agg_kernel(q_ref, edges_ref, s24_ref, p192_ref, out_ref, a_s, c_s):
    nb2 = _NBR * _NBR

    @pl.when(pl.program_id(0) == 0)
    def _build_coeffs():
        s_eff = (s24_ref[:, 0:_NBR] + s24_ref[:, _NBR:2 * _NBR]
                 + s24_ref[:, 2 * _NBR:3 * _NBR]) * (1.0 / _HEADS)
        p_mean = (p192_ref[:, 0:nb2] + p192_ref[:, nb2:2 * nb2]
                  + p192_ref[:, 2 * nb2:3 * nb2]) * (1.0 / _HEADS)
        ij = jax.lax.broadcasted_iota(jnp.int32, (_N, nb2), 1)
        tri = ((ij // _NBR) < (ij % _NBR)).astype(jnp.float32)
        p_eff = p_mean * tri
        l_i = jax.lax.broadcasted_iota(jnp.int32, (nb2, _NBR), 0)
        k_i = jax.lax.broadcasted_iota(jnp.int32, (nb2, _NBR), 1)
        rmask = ((l_i // _NBR) == k_i).astype(jnp.float32)
        cmask = ((l_i % _NBR) == k_i).astype(jnp.float32)
        rowsum = jnp.dot(p_eff, rmask, preferred_element_type=jnp.float32)
        colsum = jnp.dot(p_eff, cmask, preferred_element_type=jnp.float32)
        w = s_eff + 0.5 * (rowsum + colsum)
        m_iota = jax.lax.broadcasted_iota(jnp.int32, (_N, _N), 1)
        acc_a = (edges_ref[:, 0:1] == m_iota).astype(jnp.float32)
        for k in range(_NBR):
            hit = (edges_ref[:, k + 1:k + 2] == m_iota).astype(jnp.float32)
            acc_a = acc_a + w[:, k:k + 1] * hit
        a_s[...] = acc_a
        c_iota = jax.lax.broadcasted_iota(jnp.int32, (_N, _N * _N), 1)
        acc_c = jnp.zeros((_N, _N * _N), jnp.float32)
        for i in range(_NBR):
            for j in range(i + 1, _NBR):
                ei = edges_ref[:, i + 1:i + 2]
                ej = edges_ref[:, j + 1:j + 2]
                rowidx = jnp.minimum(ei, ej) * _N + jnp.maximum(ei, ej)
                val = p_eff[:, i * _NBR + j:i * _NBR + j + 1] * (-0.5)
                acc_c = acc_c + val * (rowidx == c_iota).astype(jnp.float32)
        c_s[...] = acc_c

    q = q_ref[...]
    t = jnp.sum(q, axis=2)
    acc = jax.lax.dot_general(t, a_s[...], (((1,), (1,)), ((), ())),
                              preferred_element_type=jnp.float32)
    cmat = c_s[...]
    for a2 in range(_N // 2):
        lo = 2 * a2
        lo8 = (lo // 8) * 8
        qa0 = q[:, lo:lo + 1, :]
        qa1 = q[:, lo + 1:lo + 2, :]
        qc = q[:, lo8:, :]
        d0 = jnp.sum(jnp.abs(qc - qa0), axis=2)
        d1 = jnp.sum(jnp.abs(qc - qa1), axis=2)
        dd = jnp.concatenate([d0, d1], axis=1)
        cslice = jnp.concatenate(
            [cmat[:, lo * _N + lo8:(lo + 1) * _N],
             cmat[:, (lo + 1) * _N + lo8:(lo + 2) * _N]], axis=1)
        acc = acc + jax.lax.dot_general(dd, cslice, (((1,), (1,)), ((), ())),
                                        preferred_element_type=jnp.float32)
    out_ref[...] = acc * (1.0 / _O)


def kernel(observation, local_edges, W1, b1, g1, be1, W2, b2, singles, pairs):
    edges = local_edges[:, 0, :].astype(jnp.int32)
    s24 = singles.reshape(_N, _HEADS * _NBR)
    p192 = pairs.reshape(_N, _HEADS * _NBR * _NBR)

    obs4 = observation.reshape(_B, _N, 1, _H)
    q = pl.pallas_call(
        _mmlp_kernel,
        grid=(_N, _B // _Bb1),
        in_specs=[
            pl.BlockSpec((_Bb1, 1, 1, _H), lambda n, bi: (bi, n, 0, 0)),
            pl.BlockSpec((1, _H, _H), lambda n, bi: (n, 0, 0)),
            pl.BlockSpec((1, 1, _H), lambda n, bi: (n, 0, 0)),
            pl.BlockSpec((1, 1, _H), lambda n, bi: (n, 0, 0)),
            pl.BlockSpec((1, 1, _H), lambda n, bi: (n, 0, 0)),
            pl.BlockSpec((1, _H, _O), lambda n, bi: (n, 0, 0)),
            pl.BlockSpec((1, 1, _O), lambda n, bi: (n, 0, 0)),
        ],
        out_specs=pl.BlockSpec((_Bb1, 1, 1, _O), lambda n, bi: (bi, n, 0, 0)),
        out_shape=jax.ShapeDtypeStruct((_B, _N, 1, _O), jnp.float32),
    )(obs4, W1, b1.reshape(_N, 1, _H), g1.reshape(_N, 1, _H),
      be1.reshape(_N, 1, _H), W2, b2.reshape(_N, 1, _O))
    q = q.reshape(_B, _N, _O)

    out = pl.pallas_call(
        _agg_kernel,
        grid=(_B // _Bb2,),
        in_specs=[
            pl.BlockSpec((_Bb2, _N, _O), lambda i: (i, 0, 0)),
            pl.BlockSpec((_N, _NBR + 1), lambda i: (0, 0)),
            pl.BlockSpec((_N, _HEADS * _NBR), lambda i: (0, 0)),
            pl.BlockSpec((_N, _HEADS * _NBR * _NBR), lambda i: (0, 0)),
        ],
        out_specs=pl.BlockSpec((_Bb2, _N), lambda i: (i, 0)),
        out_shape=jax.ShapeDtypeStruct((_B, _N), jnp.float32),
        scratch_shapes=[
            pltpu.VMEM((_N, _N), jnp.float32),
            pltpu.VMEM((_N, _N * _N), jnp.float32),
        ],
    )(q, edges, s24, p192)
    return out
